# baseline (speedup 1.0000x reference)
import jax
import jax.numpy as jnp
from jax import lax
from jax.experimental import pallas as pl
from jax.experimental.pallas import tpu as pltpu
from jax.experimental.pallas import tpu_sc as plsc

B, V, K = 8, 2, 8192
H, W = 512, 512
HW = H * W
N = V * B * HW
KV = K // 16
SLAB = 64 * W
NSLAB = HW // 2 // SLAB

_mesh = plsc.VectorSubcoreMesh(core_axis_name="c", subcore_axis_name="s")

_GATHER_DNUMS = lax.GatherDimensionNumbers(
    offset_dims=(), collapsed_slice_dims=(0,), start_index_map=(0,)
)


def _dyn_gather(vec, idx):
    return lax.gather(
        vec,
        idx[:, None],
        dimension_numbers=_GATHER_DNUMS,
        slice_sizes=(1,),
        mode=lax.GatherScatterMode.PROMISE_IN_BOUNDS,
    )


_SCRATCH_TYPES = [
    pltpu.VMEM((K,), jnp.int32),
    pltpu.VMEM((K,), jnp.int32),
    pltpu.VMEM((K,), jnp.float32),
    pltpu.VMEM((K,), jnp.int32),
    pltpu.VMEM((K,), jnp.float32),
    pltpu.VMEM((SLAB,), jnp.float32),
    pltpu.VMEM((SLAB,), jnp.float32),
    pltpu.SemaphoreType.DMA,
    pltpu.SemaphoreType.DMA,
    pltpu.SemaphoreType.DMA,
    pltpu.SemaphoreType.DMA,
]


def _fuse_body(x_hbm, y_hbm, z_hbm, in_hbm, out_hbm,
               xv, yv, zv, av, vv, slab0, slab1, si0, si1, so0, so1):
    cid = lax.axis_index("c")
    sid = lax.axis_index("s")
    p = sid
    b = lax.rem(p, B)
    v = lax.div(p, B)
    blk = (b * V + v) * K

    base = p * HW + cid * (HW // 2)
    bufs = [slab0, slab1]
    isems = [si0, si1]
    osems = [so0, so1]

    ins = [None] * NSLAB
    outs = [None] * NSLAB
    for s in range(2):
        ins[s] = pltpu.async_copy(
            in_hbm.at[pl.ds(base + s * SLAB, SLAB)], bufs[s], isems[s])

    pltpu.sync_copy(x_hbm.at[pl.ds(blk, K)], xv)
    pltpu.sync_copy(y_hbm.at[pl.ds(blk, K)], yv)
    pltpu.sync_copy(z_hbm.at[pl.ds(blk, K)], zv)

    lane = lax.iota(jnp.int32, 16)

    def prep_body(i, carry):
        sl = pl.ds(i * 16, 16)
        av[sl] = yv[sl] * W + xv[sl]
        zv[sl] = 0.5 * (1.0 / zv[sl])
        return carry

    lax.fori_loop(0, KV, prep_body, 0)

    for s in range(NSLAB):
        cur = bufs[s % 2]
        lo = cid * (HW // 2) + s * SLAB
        ins[s].wait()

        def gather_body(i, carry, lo=lo, cur=cur):
            sl = pl.ds(i * 16, 16)
            l = av[sl] - lo
            m = (l >= 0) & (l < SLAB)
            off = jnp.where(m, l, 0)
            orig = plsc.load_gather(cur, [off], mask=m)
            vv[sl] = zv[sl] + 0.5 * orig
            return carry

        lax.fori_loop(0, KV, gather_body, 0)

        def scatter_body(i, carry, lo=lo, cur=cur):
            sl = pl.ds(i * 16, 16)
            l = av[sl] - lo
            m = (l >= 0) & (l < SLAB)
            off = jnp.where(m, l, 0)
            val = vv[sl]
            plsc.store_scatter(cur, [off], val, mask=m)
            rb = plsc.load_gather(cur, [off], mask=m)
            bad = m & (rb != val)

            @pl.when(jnp.any(bad))
            def _fix():
                loser = l != l
                for sft in range(1, 16):
                    perm = lax.bitwise_and(lane + sft, 15)
                    loser = loser | ((_dyn_gather(l, perm) == l)
                                     & (lane < (16 - sft)))
                plsc.store_scatter(cur, [off], val, mask=m & (~loser))

            return carry

        lax.fori_loop(0, KV, scatter_body, 0)

        outs[s] = pltpu.async_copy(
            cur, out_hbm.at[pl.ds(base + s * SLAB, SLAB)], osems[s % 2])
        if s + 2 < NSLAB:
            outs[s].wait()
            ins[s + 2] = pltpu.async_copy(
                in_hbm.at[pl.ds(base + (s + 2) * SLAB, SLAB)],
                bufs[s % 2], isems[s % 2])

    for s in range(NSLAB - 2, NSLAB):
        outs[s].wait()


_fuse = pl.kernel(
    _fuse_body,
    out_type=jax.ShapeDtypeStruct((N,), jnp.float32),
    mesh=_mesh,
    scratch_types=_SCRATCH_TYPES,
    compiler_params=pltpu.CompilerParams(needs_layout_passes=False),
)


def kernel(mk_xy, mk_depth, fullres_disps):
    x = mk_xy[..., 0].reshape(-1)
    y = mk_xy[..., 1].reshape(-1)
    z = mk_depth.reshape(-1)
    in_flat = fullres_disps.reshape(N)
    out = _fuse(x, y, z, in_flat)
    return out.reshape(V * B, 1, H, W)


# speedup vs baseline: 1.8059x; 1.8059x over previous
import jax
import jax.numpy as jnp
from jax import lax
from jax.experimental import pallas as pl
from jax.experimental.pallas import tpu as pltpu
from jax.experimental.pallas import tpu_sc as plsc

B, V, K = 8, 2, 8192
H, W = 512, 512
HW = H * W
N = V * B * HW
KV = K // 16
SLAB = 64 * W
NSLAB = HW // 2 // SLAB
SENTINEL = 1 << 23

_mesh = plsc.VectorSubcoreMesh(core_axis_name="c", subcore_axis_name="s")

_GATHER_DNUMS = lax.GatherDimensionNumbers(
    offset_dims=(), collapsed_slice_dims=(0,), start_index_map=(0,)
)


def _dyn_gather(vec, idx):
    return lax.gather(
        vec,
        idx[:, None],
        dimension_numbers=_GATHER_DNUMS,
        slice_sizes=(1,),
        mode=lax.GatherScatterMode.PROMISE_IN_BOUNDS,
    )


_SCRATCH_TYPES = [
    pltpu.VMEM((K,), jnp.int32),
    pltpu.VMEM((K,), jnp.int32),
    pltpu.VMEM((K,), jnp.float32),
    pltpu.VMEM((K,), jnp.int32),
    pltpu.VMEM((K,), jnp.float32),
    pltpu.VMEM((SLAB,), jnp.float32),
    pltpu.VMEM((SLAB,), jnp.float32),
    pltpu.SemaphoreType.DMA,
    pltpu.SemaphoreType.DMA,
    pltpu.SemaphoreType.DMA,
    pltpu.SemaphoreType.DMA,
]


def _fuse_body(x_hbm, y_hbm, z_hbm, in_hbm, out_hbm,
               xv, yv, zv, av, vv, slab0, slab1, si0, si1, so0, so1):
    cid = lax.axis_index("c")
    sid = lax.axis_index("s")
    p = sid
    b = lax.rem(p, B)
    v = lax.div(p, B)
    blk = (b * V + v) * K

    base = p * HW + cid * (HW // 2)
    bufs = [slab0, slab1]
    isems = [si0, si1]
    osems = [so0, so1]

    ins = [None] * NSLAB
    outs = [None] * NSLAB
    for s in range(2):
        ins[s] = pltpu.async_copy(
            in_hbm.at[pl.ds(base + s * SLAB, SLAB)], bufs[s], isems[s])

    pltpu.sync_copy(x_hbm.at[pl.ds(blk, K)], xv)
    pltpu.sync_copy(y_hbm.at[pl.ds(blk, K)], yv)
    pltpu.sync_copy(z_hbm.at[pl.ds(blk, K)], zv)

    lane = lax.iota(jnp.int32, 16)
    pnext = jnp.minimum(lane + 1, 15)

    @plsc.parallel_loop(0, KV, unroll=2)
    def _prep(i):
        sl = pl.ds(i * 16, 16)
        l = yv[sl] * W + xv[sl]
        key = lax.shift_left(l, 4) | lane
        ks, zs = plsc.sort_key_val(key, zv[sl])
        a = lax.shift_right_logical(ks, 4)
        nxt = lax.shift_right_logical(_dyn_gather(ks, pnext), 4)
        dup = (nxt == a) & (lane < 15)
        av[sl] = jnp.where(dup, SENTINEL, a)
        zv[sl] = 0.5 * (1.0 / zs)

    for s in range(NSLAB):
        cur = bufs[s % 2]
        lo = cid * (HW // 2) + s * SLAB
        ins[s].wait()

        @plsc.parallel_loop(0, KV, unroll=4)
        def _gather(i, lo=lo, cur=cur):
            sl = pl.ds(i * 16, 16)
            l = av[sl] - lo
            m = (l >= 0) & (l < SLAB)
            off = jnp.where(m, l, 0)
            orig = plsc.load_gather(cur, [off], mask=m)
            vv[sl] = zv[sl] + 0.5 * orig

        def scatter_body(t, carry, lo=lo, cur=cur):
            for u in range(4):
                i = t * 4 + u
                sl = pl.ds(i * 16, 16)
                l = av[sl] - lo
                m = (l >= 0) & (l < SLAB)
                off = jnp.where(m, l, 0)
                plsc.store_scatter(cur, [off], vv[sl], mask=m)
            return carry

        lax.fori_loop(0, KV // 4, scatter_body, 0)

        outs[s] = pltpu.async_copy(
            cur, out_hbm.at[pl.ds(base + s * SLAB, SLAB)], osems[s % 2])
        if s + 2 < NSLAB:
            outs[s].wait()
            ins[s + 2] = pltpu.async_copy(
                in_hbm.at[pl.ds(base + (s + 2) * SLAB, SLAB)],
                bufs[s % 2], isems[s % 2])

    for s in range(NSLAB - 2, NSLAB):
        outs[s].wait()


_fuse = pl.kernel(
    _fuse_body,
    out_type=jax.ShapeDtypeStruct((N,), jnp.float32),
    mesh=_mesh,
    scratch_types=_SCRATCH_TYPES,
    compiler_params=pltpu.CompilerParams(needs_layout_passes=False),
)


def kernel(mk_xy, mk_depth, fullres_disps):
    x = mk_xy[..., 0].reshape(-1)
    y = mk_xy[..., 1].reshape(-1)
    z = mk_depth.reshape(-1)
    in_flat = fullres_disps.reshape(N)
    out = _fuse(x, y, z, in_flat)
    return out.reshape(V * B, 1, H, W)


# speedup vs baseline: 2.5031x; 1.3861x over previous
import jax
import jax.numpy as jnp
from jax import lax
from jax.experimental import pallas as pl
from jax.experimental.pallas import tpu as pltpu
from jax.experimental.pallas import tpu_sc as plsc

B, V, K = 8, 2, 8192
H, W = 512, 512
HW = H * W
N = V * B * HW
KV = K // 16
SLAB = 64 * W
NSLAB = HW // 2 // SLAB
SENTINEL = 1 << 23
SROWS = 64

_mesh = plsc.VectorSubcoreMesh(core_axis_name="c", subcore_axis_name="s")

_GATHER_DNUMS = lax.GatherDimensionNumbers(
    offset_dims=(), collapsed_slice_dims=(0,), start_index_map=(0,)
)


def _dyn_gather(vec, idx):
    return lax.gather(
        vec,
        idx[:, None],
        dimension_numbers=_GATHER_DNUMS,
        slice_sizes=(1,),
        mode=lax.GatherScatterMode.PROMISE_IN_BOUNDS,
    )


_SCRATCH_TYPES = [
    pltpu.VMEM((K,), jnp.int32),
    pltpu.VMEM((K,), jnp.int32),
    pltpu.VMEM((K,), jnp.float32),
    pltpu.VMEM((K,), jnp.int32),
    pltpu.VMEM((K,), jnp.float32),
    pltpu.VMEM((SROWS, W), jnp.float32),
    pltpu.VMEM((SROWS, W), jnp.float32),
    pltpu.SemaphoreType.DMA,
    pltpu.SemaphoreType.DMA,
    pltpu.SemaphoreType.DMA,
    pltpu.SemaphoreType.DMA,
]


def _fuse_body(x_hbm, y_hbm, z_hbm, in_hbm, out_hbm,
               xv, yv, zv, av, vv, slab0, slab1, si0, si1, so0, so1):
    cid = lax.axis_index("c")
    sid = lax.axis_index("s")
    p = sid
    b = lax.rem(p, B)
    v = lax.div(p, B)
    blk = (b * V + v) * K

    row_base = cid * (H // 2)
    bufs = [slab0, slab1]
    isems = [si0, si1]
    osems = [so0, so1]

    ins = [None] * NSLAB
    outs = [None] * NSLAB
    for s in range(2):
        ins[s] = pltpu.async_copy(
            in_hbm.at[p, 0, pl.ds(row_base + s * SROWS, SROWS)],
            bufs[s], isems[s])

    pltpu.sync_copy(x_hbm.at[pl.ds(blk, K)], xv)
    pltpu.sync_copy(y_hbm.at[pl.ds(blk, K)], yv)
    pltpu.sync_copy(z_hbm.at[pl.ds(blk, K)], zv)

    lane = lax.iota(jnp.int32, 16)
    pnext = jnp.minimum(lane + 1, 15)

    @plsc.parallel_loop(0, KV, unroll=2)
    def _prep(i):
        sl = pl.ds(i * 16, 16)
        l = yv[sl] * W + xv[sl]
        key = lax.shift_left(l, 4) | lane
        ks, zs = plsc.sort_key_val(key, zv[sl])
        a = lax.shift_right_logical(ks, 4)
        nxt = lax.shift_right_logical(_dyn_gather(ks, pnext), 4)
        dup = (nxt == a) & (lane < 15)
        av[sl] = jnp.where(dup, SENTINEL, a)
        zv[sl] = 0.5 * (1.0 / zs)

    for s in range(NSLAB):
        cur = bufs[s % 2]
        row0 = row_base + s * SROWS
        ins[s].wait()

        @plsc.parallel_loop(0, KV, unroll=4)
        def _gather(i, row0=row0, cur=cur):
            sl = pl.ds(i * 16, 16)
            a = av[sl]
            lr = lax.shift_right_logical(a, 9) - row0
            m = (lr >= 0) & (lr < SROWS)
            lrc = jnp.where(m, lr, 0)
            col = lax.bitwise_and(a, W - 1)
            orig = plsc.load_gather(cur, [lrc, col], mask=m)
            vv[sl] = zv[sl] + 0.5 * orig

        def scatter_body(t, carry, row0=row0, cur=cur):
            for u in range(4):
                i = t * 4 + u
                sl = pl.ds(i * 16, 16)
                a = av[sl]
                lr = lax.shift_right_logical(a, 9) - row0
                m = (lr >= 0) & (lr < SROWS)
                lrc = jnp.where(m, lr, 0)
                col = lax.bitwise_and(a, W - 1)
                plsc.store_scatter(cur, [lrc, col], vv[sl], mask=m)
            return carry

        lax.fori_loop(0, KV // 4, scatter_body, 0)

        outs[s] = pltpu.async_copy(
            cur, out_hbm.at[p, 0, pl.ds(row0, SROWS)], osems[s % 2])
        if s + 2 < NSLAB:
            outs[s].wait()
            ins[s + 2] = pltpu.async_copy(
                in_hbm.at[p, 0, pl.ds(row_base + (s + 2) * SROWS, SROWS)],
                bufs[s % 2], isems[s % 2])

    for s in range(NSLAB - 2, NSLAB):
        outs[s].wait()


_fuse = pl.kernel(
    _fuse_body,
    out_type=jax.ShapeDtypeStruct((V * B, 1, H, W), jnp.float32),
    mesh=_mesh,
    scratch_types=_SCRATCH_TYPES,
    compiler_params=pltpu.CompilerParams(
        needs_layout_passes=False, use_tc_tiling_on_sc=True),
)


def kernel(mk_xy, mk_depth, fullres_disps):
    x = mk_xy[..., 0].reshape(-1)
    y = mk_xy[..., 1].reshape(-1)
    z = mk_depth.reshape(-1)
    return _fuse(x, y, z, fullres_disps)


# speedup vs baseline: 3.1575x; 1.2614x over previous
import jax
import jax.numpy as jnp
from jax import lax
from jax.experimental import pallas as pl
from jax.experimental.pallas import tpu as pltpu
from jax.experimental.pallas import tpu_sc as plsc

B, V, K = 8, 2, 8192
H, W = 512, 512
HW = H * W
N = V * B * HW
KV = K // 16
SLAB = 64 * W
NSLAB = HW // 2 // SLAB
SENTINEL = 1 << 23
SROWS = 64

_mesh = plsc.VectorSubcoreMesh(core_axis_name="c", subcore_axis_name="s")

_GATHER_DNUMS = lax.GatherDimensionNumbers(
    offset_dims=(), collapsed_slice_dims=(0,), start_index_map=(0,)
)


def _dyn_gather(vec, idx):
    return lax.gather(
        vec,
        idx[:, None],
        dimension_numbers=_GATHER_DNUMS,
        slice_sizes=(1,),
        mode=lax.GatherScatterMode.PROMISE_IN_BOUNDS,
    )


ARENA = K + 16 * NSLAB + 16

_SCRATCH_TYPES = [
    pltpu.VMEM((K,), jnp.int32),
    pltpu.VMEM((K,), jnp.int32),
    pltpu.VMEM((K,), jnp.float32),
    pltpu.VMEM((K,), jnp.int32),
    pltpu.VMEM((ARENA,), jnp.int32),
    pltpu.VMEM((ARENA,), jnp.float32),
    pltpu.VMEM((ARENA,), jnp.float32),
    pltpu.VMEM((SROWS, W), jnp.float32),
    pltpu.VMEM((SROWS, W), jnp.float32),
    pltpu.SemaphoreType.DMA,
    pltpu.SemaphoreType.DMA,
    pltpu.SemaphoreType.DMA,
    pltpu.SemaphoreType.DMA,
]


def _fuse_body(x_hbm, y_hbm, z_hbm, in_hbm, out_hbm,
               xv, yv, zv, av, aa, za, vv, slab0, slab1, si0, si1, so0, so1):
    cid = lax.axis_index("c")
    sid = lax.axis_index("s")
    p = sid
    b = lax.rem(p, B)
    v = lax.div(p, B)
    blk = (b * V + v) * K

    row_base = cid * (H // 2)
    bufs = [slab0, slab1]
    isems = [si0, si1]
    osems = [so0, so1]

    ins = [None] * NSLAB
    outs = [None] * NSLAB
    for s in range(2):
        ins[s] = pltpu.async_copy(
            in_hbm.at[p, 0, pl.ds(row_base + s * SROWS, SROWS)],
            bufs[s], isems[s])

    pltpu.sync_copy(x_hbm.at[pl.ds(blk, K)], xv)
    pltpu.sync_copy(y_hbm.at[pl.ds(blk, K)], yv)
    pltpu.sync_copy(z_hbm.at[pl.ds(blk, K)], zv)

    lane = lax.iota(jnp.int32, 16)
    pnext = jnp.minimum(lane + 1, 15)
    rb0 = cid * NSLAB
    zero = jnp.int32(0)

    @plsc.parallel_loop(0, KV, unroll=2, carry=(zero, zero, zero, zero))
    def _prep(i, cnt):
        sl = pl.ds(i * 16, 16)
        l = yv[sl] * W + xv[sl]
        key = lax.shift_left(l, 4) | lane
        ks, zs = plsc.sort_key_val(key, zv[sl])
        a = lax.shift_right_logical(ks, 4)
        nxt = lax.shift_right_logical(_dyn_gather(ks, pnext), 4)
        dup = (nxt == a) & (lane < 15)
        a = jnp.where(dup, SENTINEL, a)
        av[sl] = a
        zv[sl] = 0.5 * (1.0 / zs)
        sb = lax.shift_right_logical(a, 15) - rb0
        return tuple(
            cnt[t] + jnp.sum((sb == t).astype(jnp.int32))
            for t in range(NSLAB))

    cnts = _prep
    offs = [zero] * NSLAB
    for t in range(1, NSLAB):
        offs[t] = lax.bitwise_and(offs[t - 1] + cnts[t - 1] + 15,
                                  jnp.int32(-16))

    @plsc.parallel_loop(0, ARENA // 16, unroll=4)
    def _fill(i):
        aa[pl.ds(i * 16, 16)] = jnp.full((16,), SENTINEL, jnp.int32)

    def compress_body(i, off):
        sl = pl.ds(i * 16, 16)
        a = av[sl]
        zz = zv[sl]
        sb = lax.shift_right_logical(a, 15) - rb0
        new = []
        for t in range(NSLAB):
            m = sb == t
            plsc.store_compressed(aa.at[pl.ds(off[t], 16)], a, mask=m)
            plsc.store_compressed(za.at[pl.ds(off[t], 16)], zz, mask=m)
            new.append(off[t] + jnp.sum(m.astype(jnp.int32)))
        return tuple(new)

    ends = lax.fori_loop(0, KV, compress_body, tuple(offs))
    nvs = [lax.shift_right_logical(ends[t] - offs[t] + 15, 4)
           for t in range(NSLAB)]

    for s in range(NSLAB):
        cur = bufs[s % 2]
        row0 = row_base + s * SROWS
        off_s = offs[s]
        nv_s = nvs[s]
        ins[s].wait()

        def gather_body(j, carry, row0=row0, cur=cur, off_s=off_s):
            sl = pl.ds(off_s + j * 16, 16)
            a = aa[sl]
            lr = lax.shift_right_logical(a, 9) - row0
            m = (lr >= 0) & (lr < SROWS)
            lrc = jnp.where(m, lr, 0)
            col = lax.bitwise_and(a, W - 1)
            orig = plsc.load_gather(cur, [lrc, col], mask=m)
            vv[sl] = za[sl] + 0.5 * orig
            return carry

        lax.fori_loop(0, nv_s, gather_body, 0)

        def scatter_body(j, carry, row0=row0, cur=cur, off_s=off_s):
            sl = pl.ds(off_s + j * 16, 16)
            a = aa[sl]
            lr = lax.shift_right_logical(a, 9) - row0
            m = (lr >= 0) & (lr < SROWS)
            lrc = jnp.where(m, lr, 0)
            col = lax.bitwise_and(a, W - 1)
            plsc.store_scatter(cur, [lrc, col], vv[sl], mask=m)
            return carry

        lax.fori_loop(0, nv_s, scatter_body, 0)

        outs[s] = pltpu.async_copy(
            cur, out_hbm.at[p, 0, pl.ds(row0, SROWS)], osems[s % 2])
        if s + 2 < NSLAB:
            outs[s].wait()
            ins[s + 2] = pltpu.async_copy(
                in_hbm.at[p, 0, pl.ds(row_base + (s + 2) * SROWS, SROWS)],
                bufs[s % 2], isems[s % 2])

    for s in range(NSLAB - 2, NSLAB):
        outs[s].wait()


_fuse = pl.kernel(
    _fuse_body,
    out_type=jax.ShapeDtypeStruct((V * B, 1, H, W), jnp.float32),
    mesh=_mesh,
    scratch_types=_SCRATCH_TYPES,
    compiler_params=pltpu.CompilerParams(
        needs_layout_passes=False, use_tc_tiling_on_sc=True),
)


def kernel(mk_xy, mk_depth, fullres_disps):
    x = mk_xy[..., 0].reshape(-1)
    y = mk_xy[..., 1].reshape(-1)
    z = mk_depth.reshape(-1)
    return _fuse(x, y, z, fullres_disps)


# speedup vs baseline: 3.1729x; 1.0049x over previous
import jax
import jax.numpy as jnp
from jax import lax
from jax.experimental import pallas as pl
from jax.experimental.pallas import tpu as pltpu
from jax.experimental.pallas import tpu_sc as plsc

B, V, K = 8, 2, 8192
H, W = 512, 512
HW = H * W
N = V * B * HW
KV = K // 16
SLAB = 64 * W
NSLAB = HW // 2 // SLAB
SENTINEL = 1 << 23
SROWS = 64

_mesh = plsc.VectorSubcoreMesh(core_axis_name="c", subcore_axis_name="s")

_GATHER_DNUMS = lax.GatherDimensionNumbers(
    offset_dims=(), collapsed_slice_dims=(0,), start_index_map=(0,)
)


def _dyn_gather(vec, idx):
    return lax.gather(
        vec,
        idx[:, None],
        dimension_numbers=_GATHER_DNUMS,
        slice_sizes=(1,),
        mode=lax.GatherScatterMode.PROMISE_IN_BOUNDS,
    )


ARENA = K + 16 * NSLAB + 16

_SCRATCH_TYPES = [
    pltpu.VMEM((K,), jnp.int32),
    pltpu.VMEM((K,), jnp.int32),
    pltpu.VMEM((K,), jnp.float32),
    pltpu.VMEM((K,), jnp.int32),
    pltpu.VMEM((ARENA,), jnp.int32),
    pltpu.VMEM((ARENA,), jnp.float32),
    pltpu.VMEM((ARENA,), jnp.float32),
    pltpu.VMEM((SROWS, W), jnp.float32),
    pltpu.VMEM((SROWS, W), jnp.float32),
    pltpu.SemaphoreType.DMA,
    pltpu.SemaphoreType.DMA,
    pltpu.SemaphoreType.DMA,
    pltpu.SemaphoreType.DMA,
]


def _fuse_body(x_hbm, y_hbm, z_hbm, in_hbm, out_hbm,
               xv, yv, zv, av, aa, za, vv, slab0, slab1, si0, si1, so0, so1):
    cid = lax.axis_index("c")
    sid = lax.axis_index("s")
    p = sid
    b = lax.rem(p, B)
    v = lax.div(p, B)
    blk = (b * V + v) * K

    row_base = cid * (H // 2)
    bufs = [slab0, slab1]
    isems = [si0, si1]
    osems = [so0, so1]

    ins = [None] * NSLAB
    outs = [None] * NSLAB
    for s in range(2):
        ins[s] = pltpu.async_copy(
            in_hbm.at[p, 0, pl.ds(row_base + s * SROWS, SROWS)],
            bufs[s], isems[s])

    pltpu.sync_copy(x_hbm.at[pl.ds(blk, K)], xv)
    pltpu.sync_copy(y_hbm.at[pl.ds(blk, K)], yv)
    pltpu.sync_copy(z_hbm.at[pl.ds(blk, K)], zv)

    lane = lax.iota(jnp.int32, 16)
    pnext = jnp.minimum(lane + 1, 15)
    rb0 = cid * NSLAB
    zero = jnp.int32(0)

    @plsc.parallel_loop(0, KV, unroll=4, carry=(zero, zero, zero, zero))
    def _prep(i, cnt):
        sl = pl.ds(i * 16, 16)
        l = yv[sl] * W + xv[sl]
        key = lax.shift_left(l, 4) | lane
        ks, zs = plsc.sort_key_val(key, zv[sl])
        a = lax.shift_right_logical(ks, 4)
        nxt = lax.shift_right_logical(_dyn_gather(ks, pnext), 4)
        dup = (nxt == a) & (lane < 15)
        a = jnp.where(dup, SENTINEL, a)
        av[sl] = a
        zv[sl] = 0.5 * (1.0 / zs)
        sb = lax.shift_right_logical(a, 15) - rb0
        return tuple(
            cnt[t] + jnp.sum((sb == t).astype(jnp.int32))
            for t in range(NSLAB))

    cnts = _prep
    offs = [zero] * NSLAB
    for t in range(1, NSLAB):
        offs[t] = lax.bitwise_and(offs[t - 1] + cnts[t - 1] + 15,
                                  jnp.int32(-16))

    sent16 = jnp.full((16,), SENTINEL, jnp.int32)
    for t in range(NSLAB):
        aa[pl.ds(offs[t] + cnts[t], 16)] = sent16

    def compress_body(i, off):
        sl = pl.ds(i * 16, 16)
        a = av[sl]
        zz = zv[sl]
        sb = lax.shift_right_logical(a, 15) - rb0
        new = []
        for t in range(NSLAB):
            m = sb == t
            plsc.store_compressed(aa.at[pl.ds(off[t], 16)], a, mask=m)
            plsc.store_compressed(za.at[pl.ds(off[t], 16)], zz, mask=m)
            new.append(off[t] + jnp.sum(m.astype(jnp.int32)))
        return tuple(new)

    ends = lax.fori_loop(0, KV, compress_body, tuple(offs))
    nvs = [lax.shift_right_logical(ends[t] - offs[t] + 15, 4)
           for t in range(NSLAB)]

    for s in range(NSLAB):
        cur = bufs[s % 2]
        row0 = row_base + s * SROWS
        off_s = offs[s]
        nv_s = nvs[s]
        ins[s].wait()

        def gather_body(j, carry, row0=row0, cur=cur, off_s=off_s):
            sl = pl.ds(off_s + j * 16, 16)
            a = aa[sl]
            lr = lax.shift_right_logical(a, 9) - row0
            m = (lr >= 0) & (lr < SROWS)
            lrc = jnp.where(m, lr, 0)
            col = lax.bitwise_and(a, W - 1)
            orig = plsc.load_gather(cur, [lrc, col], mask=m)
            vv[sl] = za[sl] + 0.5 * orig
            return carry

        lax.fori_loop(0, nv_s, gather_body, 0)

        def scatter_body(j, carry, row0=row0, cur=cur, off_s=off_s):
            sl = pl.ds(off_s + j * 16, 16)
            a = aa[sl]
            lr = lax.shift_right_logical(a, 9) - row0
            m = (lr >= 0) & (lr < SROWS)
            lrc = jnp.where(m, lr, 0)
            col = lax.bitwise_and(a, W - 1)
            plsc.store_scatter(cur, [lrc, col], vv[sl], mask=m)
            return carry

        lax.fori_loop(0, nv_s, scatter_body, 0)

        outs[s] = pltpu.async_copy(
            cur, out_hbm.at[p, 0, pl.ds(row0, SROWS)], osems[s % 2])
        if s + 2 < NSLAB:
            outs[s].wait()
            ins[s + 2] = pltpu.async_copy(
                in_hbm.at[p, 0, pl.ds(row_base + (s + 2) * SROWS, SROWS)],
                bufs[s % 2], isems[s % 2])

    for s in range(NSLAB - 2, NSLAB):
        outs[s].wait()


_fuse = pl.kernel(
    _fuse_body,
    out_type=jax.ShapeDtypeStruct((V * B, 1, H, W), jnp.float32),
    mesh=_mesh,
    scratch_types=_SCRATCH_TYPES,
    compiler_params=pltpu.CompilerParams(
        needs_layout_passes=False, use_tc_tiling_on_sc=True),
)


def kernel(mk_xy, mk_depth, fullres_disps):
    x = mk_xy[..., 0].reshape(-1)
    y = mk_xy[..., 1].reshape(-1)
    z = mk_depth.reshape(-1)
    return _fuse(x, y, z, fullres_disps)
